# Initial kernel scaffold; baseline (speedup 1.0000x reference)
import jax
import jax.numpy as jnp
from jax.experimental import pallas as pl


def kernel(idx, e1_w, e2_w, l1_w, l1_b, l2_w, l2_b):
    raise NotImplementedError("write your pallas kernel here")


# speedup vs baseline: 6.9298x; 6.9298x over previous
import functools

import jax
import jax.numpy as jnp
from jax import lax
from jax.experimental import pallas as pl
from jax.experimental.pallas import tpu as pltpu
from jax.experimental.pallas import tpu_sc as plsc

_ALPHA = 3.0
_K = 64
_N = 4096
_D = 512
_BR = 256
_NC = 2
_NS = 16
_BPW = _N // (_NC * _NS)


def _sc_gather(idx, t1, t2):
    mesh = plsc.VectorSubcoreMesh(core_axis_name="c", subcore_axis_name="s")

    @functools.partial(
        pl.kernel,
        mesh=mesh,
        out_type=[
            jax.ShapeDtypeStruct((_N, _D), jnp.float32),
            jax.ShapeDtypeStruct((_N, _D), jnp.float32),
        ],
        scratch_types=[
            pltpu.VMEM((_BPW,), jnp.int32),
            pltpu.VMEM((_BPW, _D), jnp.float32),
            pltpu.SemaphoreType.DMA,
        ],
    )
    def gather_k(idx_hbm, t1_hbm, t2_hbm, o1_hbm, o2_hbm, idx_v, rows_v, sem):
        wid = lax.axis_index("s") * _NC + lax.axis_index("c")
        base = wid * _BPW
        pltpu.sync_copy(idx_hbm.at[pl.ds(base, _BPW)], idx_v)
        pltpu.async_copy(t1_hbm.at[idx_v], rows_v, sem).wait()
        pltpu.sync_copy(rows_v, o1_hbm.at[pl.ds(base, _BPW)])
        pltpu.async_copy(t2_hbm.at[idx_v], rows_v, sem).wait()
        pltpu.sync_copy(rows_v, o2_hbm.at[pl.ds(base, _BPW)])

    return gather_k(idx, t1, t2)


def _mlp_body(g_ref, wT_ref, b_ref, o_ref):
    x = jnp.dot(g_ref[...], wT_ref[...], preferred_element_type=jnp.float32)
    o_ref[...] = jnp.tanh(_ALPHA * (x + b_ref[...]))


def _mlp(g, wT, b):
    return pl.pallas_call(
        _mlp_body,
        grid=(_N // _BR,),
        in_specs=[
            pl.BlockSpec((_BR, _D), lambda i: (i, 0)),
            pl.BlockSpec((_D, _D), lambda i: (0, 0)),
            pl.BlockSpec((1, _D), lambda i: (0, 0)),
        ],
        out_specs=pl.BlockSpec((_BR, _D), lambda i: (i, 0)),
        out_shape=jax.ShapeDtypeStruct((_N, _D), jnp.float32),
    )(g, wT, b)


def _adj_body(m1_ref, m2_ref, noise_ref, o_ref):
    a = lax.dot_general(
        m1_ref[...], m2_ref[...],
        (((1,), (1,)), ((), ())),
        preferred_element_type=jnp.float32,
    )
    adj = jnp.maximum(jnp.tanh(_ALPHA * a), 0.0)
    v = adj + noise_ref[...]
    vb = lax.bitcast_convert_type(v, jnp.int32)

    def hi_body(i, t):
        bit = jnp.left_shift(jnp.int32(1), jnp.int32(29) - i)
        cand = t | bit
        cnt = jnp.sum((vb >= cand).astype(jnp.int32), axis=1, keepdims=True)
        return jnp.where(cnt >= _K, cand, t)

    t = lax.fori_loop(0, 30, hi_body, jnp.zeros((_BR, 1), jnp.int32))

    gt = vb > t
    eq = vb == t
    cntg = jnp.sum(gt.astype(jnp.int32), axis=1, keepdims=True)
    need = _K - cntg
    col = lax.broadcasted_iota(jnp.int32, (_BR, _N), 1)

    def ix_body(i, c):
        cand = c + jnp.left_shift(jnp.int32(1), jnp.int32(12) - i)
        f = jnp.sum((eq & (col <= cand)).astype(jnp.int32), axis=1,
                    keepdims=True)
        return jnp.where(f <= need, cand, c)

    c = lax.fori_loop(0, 13, ix_body, jnp.full((_BR, 1), -1, jnp.int32))

    mask = gt | (eq & (col <= c))
    o_ref[...] = jnp.where(mask, adj, 0.0)


def _adj_topk(m1, m2, noise):
    return pl.pallas_call(
        _adj_body,
        grid=(_N // _BR,),
        in_specs=[
            pl.BlockSpec((_BR, _D), lambda i: (i, 0)),
            pl.BlockSpec((_N, _D), lambda i: (0, 0)),
            pl.BlockSpec((_BR, _N), lambda i: (i, 0)),
        ],
        out_specs=pl.BlockSpec((_BR, _N), lambda i: (i, 0)),
        out_shape=jax.ShapeDtypeStruct((_N, _N), jnp.float32),
    )(m1, m2, noise)


def kernel(idx, e1_w, e2_w, l1_w, l1_b, l2_w, l2_b):
    idx = idx.astype(jnp.int32)
    g1, g2 = _sc_gather(idx, e1_w, e2_w)
    m1 = _mlp(g1, l1_w.T, l1_b.reshape(1, _D))
    m2 = _mlp(g2, l2_w.T, l2_b.reshape(1, _D))
    noise = jax.random.uniform(jax.random.key(1), (_N, _N),
                               dtype=jnp.float32) * 0.01
    return _adj_topk(m1, m2, noise)
